# baseline (speedup 1.0000x reference)
import functools

import jax
import jax.numpy as jnp
from jax import lax
from jax.experimental import pallas as pl
from jax.experimental.pallas import tpu as pltpu
from jax.experimental.pallas import tpu_sc as plsc

B = 4096
F = 26
ND = 13
VOCAB = 100000
D = 64
L = 16

NC = 2
NS = 16
NW = NC * NS
BPW = B // NW
GROUP = 16
NGROUPS = BPW // GROUP
ROWS_PER_GROUP = GROUP * F
IDX_ROW = 104
IDX_ROWS_PER_GROUP = ROWS_PER_GROUP // IDX_ROW
NIDXROWS = BPW * F // IDX_ROW


def _fm_body(idx_hbm, dense_hbm, const_hbm, table_hbm, slw_hbm, out_hbm,
             idx_v, rows_v, slw_v, dense_v, const_v, out_v, sems, sems2):
    wid = lax.axis_index("s") * NC + lax.axis_index("c")
    pltpu.sync_copy(idx_hbm.at[wid], idx_v)
    lane = lax.iota(jnp.int32, L)
    slw_idx = lane * F

    def issue_gathers(g):
        buf = g % 2
        copies = []
        for j in range(IDX_ROWS_PER_GROUP):
            row = g * IDX_ROWS_PER_GROUP + j
            copies.append(pltpu.async_copy(
                table_hbm.at[idx_v.at[row]],
                rows_v.at[buf].at[pl.ds(j * IDX_ROW, IDX_ROW)], sems[buf]))
            copies.append(pltpu.async_copy(
                slw_hbm.at[idx_v.at[row]],
                slw_v.at[buf].at[pl.ds(j * IDX_ROW, IDX_ROW)], sems2[buf]))
        return copies

    inflight = issue_gathers(0)
    pltpu.sync_copy(dense_hbm.at[wid], dense_v)
    pltpu.sync_copy(const_hbm, const_v)
    w_vec = const_v[0, :]
    bconst_vec = const_v[1, :]

    for g in range(NGROUPS):
        buf = g % 2
        for c in inflight:
            c.wait()
        if g + 1 < NGROUPS:
            inflight = issue_gathers(g + 1)

        ssum_vec = jnp.zeros((L,), jnp.float32)
        for f in range(F):
            ssum_vec = ssum_vec + plsc.load_gather(slw_v.at[buf],
                                                   [slw_idx + f])

        def item_body(i, out_vec):
            base = i * F
            s = [jnp.zeros((L,), jnp.float32) for _ in range(4)]
            q = [jnp.zeros((L,), jnp.float32) for _ in range(4)]
            for f in range(F):
                for j in range(4):
                    r = rows_v[buf, base + f, pl.ds(j * L, L)]
                    s[j] = s[j] + r
                    q[j] = q[j] + r * r
            dv = dense_v[g * GROUP + i, :]
            tv = (0.5 * (s[0] * s[0] + s[1] * s[1] + s[2] * s[2] + s[3] * s[3]
                         - (q[0] + q[1] + q[2] + q[3]))
                  + dv * w_vec)
            for sh in (8, 4, 2, 1):
                tv = tv + tv.at[lane ^ sh].get(mode="promise_in_bounds")
            return jnp.where(lane == i, out_vec + tv, out_vec)

        out_vec = lax.fori_loop(0, GROUP, item_body, ssum_vec + bconst_vec)
        out_v[pl.ds(g * GROUP, GROUP)] = out_vec

    pltpu.sync_copy(out_v, out_hbm.at[pl.ds(wid * BPW, BPW)])


_SCRATCH = [
    pltpu.VMEM((NIDXROWS, IDX_ROW), jnp.int32),
    pltpu.VMEM((2, ROWS_PER_GROUP, 2 * D), jnp.float32),
    pltpu.VMEM((2, ROWS_PER_GROUP), jnp.float32),
    pltpu.VMEM((BPW, L), jnp.float32),
    pltpu.VMEM((2, L), jnp.float32),
    pltpu.VMEM((BPW,), jnp.float32),
    [pltpu.SemaphoreType.DMA, pltpu.SemaphoreType.DMA],
    [pltpu.SemaphoreType.DMA, pltpu.SemaphoreType.DMA],
]


def _build():
    return pl.kernel(
        _fm_body,
        out_type=jax.ShapeDtypeStruct((B,), jnp.float32),
        mesh=plsc.VectorSubcoreMesh(core_axis_name="c", subcore_axis_name="s",
                                    num_cores=NC, num_subcores=NS),
        compiler_params=pltpu.CompilerParams(needs_layout_passes=False,
                                             use_tc_tiling_on_sc=False),
        scratch_types=_SCRATCH,
    )


def _prep(dense, sparse, sparse_linear_w, sparse_embedding_w,
          dense_linear_w, dense_linear_b, bias):
    idx = sparse.astype(jnp.int32).reshape(NW, NIDXROWS, IDX_ROW)
    dense_p = jnp.pad(dense, ((0, 0), (0, L - ND))).reshape(NW, BPW, L)
    w_row = jnp.pad(dense_linear_w.reshape(ND), (0, L - ND))
    bconst = (dense_linear_b + bias).astype(jnp.float32)
    const = jnp.stack([w_row, jnp.broadcast_to(bconst, (L,))])
    slw = sparse_linear_w.reshape(VOCAB)
    table_p = jnp.pad(sparse_embedding_w, ((0, 0), (0, D)))
    return idx, dense_p, const, table_p, slw


def kernel(dense, sparse, sparse_linear_w, sparse_embedding_w,
           dense_linear_w, dense_linear_b, bias):
    args = _prep(dense, sparse, sparse_linear_w, sparse_embedding_w,
                 dense_linear_w, dense_linear_b, bias)
    return _build()(*args)


# speedup vs baseline: 1.0260x; 1.0260x over previous
import functools

import jax
import jax.numpy as jnp
from jax import lax
from jax.experimental import pallas as pl
from jax.experimental.pallas import tpu as pltpu
from jax.experimental.pallas import tpu_sc as plsc

B = 4096
F = 26
ND = 13
VOCAB = 100000
D = 64
L = 16

NC = 2
NS = 16
NW = NC * NS
BPW = B // NW
GROUP = 16
NGROUPS = BPW // GROUP
ROWS_PER_GROUP = GROUP * F
IDX_ROW = 104
IDX_ROWS_PER_GROUP = ROWS_PER_GROUP // IDX_ROW
NIDXROWS = BPW * F // IDX_ROW


def _fm_body(idx_hbm, dense_hbm, const_hbm, table_hbm, slw_hbm, out_hbm,
             idx_v, rows_v, slw_v, dense_v, const_v, out_v, sems, sems2):
    wid = lax.axis_index("s") * NC + lax.axis_index("c")
    pltpu.sync_copy(idx_hbm.at[wid], idx_v)
    lane = lax.iota(jnp.int32, L)
    slw_idx = lane * F

    def issue_gathers(g):
        buf = g % 2
        copies = []
        for j in range(IDX_ROWS_PER_GROUP):
            row = g * IDX_ROWS_PER_GROUP + j
            copies.append(pltpu.async_copy(
                table_hbm.at[idx_v.at[row]],
                rows_v.at[buf].at[pl.ds(j * IDX_ROW, IDX_ROW)], sems[buf]))
            copies.append(pltpu.async_copy(
                slw_hbm.at[idx_v.at[row]],
                slw_v.at[buf].at[pl.ds(j * IDX_ROW, IDX_ROW)], sems2[buf]))
        return copies

    inflight = issue_gathers(0)
    pltpu.sync_copy(dense_hbm.at[wid], dense_v)
    pltpu.sync_copy(const_hbm, const_v)
    w_vec = const_v[0, :]
    bconst_vec = const_v[1, :]

    for g in range(NGROUPS):
        buf = g % 2
        for c in inflight:
            c.wait()
        if g + 1 < NGROUPS:
            inflight = issue_gathers(g + 1)

        ssum_vec = jnp.zeros((L,), jnp.float32)
        for f in range(F):
            ssum_vec = ssum_vec + plsc.load_gather(slw_v.at[buf],
                                                   [slw_idx + f])

        def item_body(i, out_vec):
            base = i * F
            s = [jnp.zeros((L,), jnp.float32) for _ in range(4)]
            q = [jnp.zeros((L,), jnp.float32) for _ in range(4)]
            for f in range(F):
                for j in range(4):
                    r = rows_v[buf, base + f, pl.ds(j * L, L)]
                    s[j] = s[j] + r
                    q[j] = q[j] + r * r
            dv = dense_v[g * GROUP + i, :]
            tv = (0.5 * (s[0] * s[0] + s[1] * s[1] + s[2] * s[2] + s[3] * s[3]
                         - (q[0] + q[1] + q[2] + q[3]))
                  + dv * w_vec)
            for sh in (8, 4, 2, 1):
                tv = tv + tv.at[lane ^ sh].get(mode="promise_in_bounds")
            return jnp.where(lane == i, out_vec + tv, out_vec)

        out_vec = lax.fori_loop(0, GROUP, item_body, ssum_vec + bconst_vec)
        out_v[pl.ds(g * GROUP, GROUP)] = out_vec

    pltpu.sync_copy(out_v, out_hbm.at[pl.ds(wid * BPW, BPW)])


_SCRATCH = [
    pltpu.VMEM((NIDXROWS, IDX_ROW), jnp.int32),
    pltpu.VMEM((2, ROWS_PER_GROUP, D), jnp.float32),
    pltpu.VMEM((2, ROWS_PER_GROUP), jnp.float32),
    pltpu.VMEM((BPW, L), jnp.float32),
    pltpu.VMEM((2, L), jnp.float32),
    pltpu.VMEM((BPW,), jnp.float32),
    [pltpu.SemaphoreType.DMA, pltpu.SemaphoreType.DMA],
    [pltpu.SemaphoreType.DMA, pltpu.SemaphoreType.DMA],
]


def _build():
    return pl.kernel(
        _fm_body,
        out_type=jax.ShapeDtypeStruct((B,), jnp.float32),
        mesh=plsc.VectorSubcoreMesh(core_axis_name="c", subcore_axis_name="s",
                                    num_cores=NC, num_subcores=NS),
        compiler_params=pltpu.CompilerParams(needs_layout_passes=False,
                                             use_tc_tiling_on_sc=False),
        scratch_types=_SCRATCH,
    )


def _prep(dense, sparse, sparse_linear_w, sparse_embedding_w,
          dense_linear_w, dense_linear_b, bias):
    idx = sparse.astype(jnp.int32).reshape(NW, NIDXROWS, IDX_ROW)
    dense_p = jnp.pad(dense, ((0, 0), (0, L - ND))).reshape(NW, BPW, L)
    w_row = jnp.pad(dense_linear_w.reshape(ND), (0, L - ND))
    bconst = (dense_linear_b + bias).astype(jnp.float32)
    const = jnp.stack([w_row, jnp.broadcast_to(bconst, (L,))])
    slw = sparse_linear_w.reshape(VOCAB)
    return idx, dense_p, const, sparse_embedding_w, slw


def kernel(dense, sparse, sparse_linear_w, sparse_embedding_w,
           dense_linear_w, dense_linear_b, bias):
    args = _prep(dense, sparse, sparse_linear_w, sparse_embedding_w,
                 dense_linear_w, dense_linear_b, bias)
    return _build()(*args)
